# baseline (speedup 1.0000x reference)
import functools

import jax
import jax.numpy as jnp
from jax.experimental import pallas as pl
from jax.experimental.pallas import tpu as pltpu
from jax.experimental.pallas import tpu_sc as plsc

_EMBED_DIM = 64
_NUM_CODES = 1024
_B = 16
_S = 576
_N_TOK = _B * _S
_TILE = _N_TOK // 2
_HROWS = _TILE // 128

_NC = 2
_NS = 16
_NW = _NC * _NS


def _idx_body(z_hbm, w_ref, idx0_ref, idx1_ref, wn_ref, zbuf_ref, zsems):
    i = pl.program_id(0)

    def _zcopy(j):
        return pltpu.make_async_copy(
            z_hbm.at[pl.ds(j * _TILE, _TILE), :],
            zbuf_ref.at[pl.ds(j * _TILE, _TILE), :], zsems[j])

    @pl.when(i == 0)
    def _init():
        _zcopy(0).start()
        _zcopy(1).start()
        w = w_ref[...]
        wn_ref[...] = w / jnp.maximum(
            jnp.sqrt(jnp.sum(w * w, axis=1, keepdims=True)), 1e-12)
        _zcopy(0).wait()

    @pl.when(i == 1)
    def _wait1():
        _zcopy(1).wait()

    zt = zbuf_ref[pl.ds(i * _TILE, _TILE), :]
    zn = zt / jnp.maximum(
        jnp.sqrt(jnp.sum(zt * zt, axis=1, keepdims=True)), 1e-12)
    scores = jax.lax.dot_general(
        zn, wn_ref[...], (((1,), (1,)), ((), ())),
        preferred_element_type=jnp.float32)
    m = jnp.max(scores, axis=1, keepdims=True)
    ids = jax.lax.broadcasted_iota(
        jnp.int32, scores.shape, 1).astype(jnp.float32)
    idx = jnp.min(jnp.where(scores == m, ids, jnp.float32(4096.0)), axis=1)
    packed = idx.astype(jnp.int32).reshape(_HROWS, 128)
    i = pl.program_id(0)

    @pl.when(i == 0)
    def _w0():
        idx0_ref[...] = packed

    @pl.when(i == 1)
    def _w1():
        idx1_ref[...] = packed


def _gather_body(w_hbm, idx0_hbm, idx1_hbm, out_hbm, idx_v, rows_v, sems):
    wid = jax.lax.axis_index("s") * _NC + jax.lax.axis_index("c")

    def loads():
        pltpu.sync_copy(idx0_hbm.at[pl.ds(wid, 1), :], idx_v.at[pl.ds(0, 1)])

        @pl.when(wid < 4)
        def _a():
            pltpu.sync_copy(idx0_hbm.at[pl.ds(wid + 32, 1), :],
                            idx_v.at[pl.ds(1, 1)])

        @pl.when(wid >= 4)
        def _b():
            pltpu.sync_copy(idx1_hbm.at[pl.ds(wid - 4, 1), :],
                            idx_v.at[pl.ds(1, 1)])

        @pl.when(wid < 8)
        def _c():
            pltpu.sync_copy(idx1_hbm.at[pl.ds(wid + 28, 1), :],
                            idx_v.at[pl.ds(2, 1)])

    def fire(j):
        pltpu.async_copy(w_hbm.at[idx_v.at[j]], rows_v.at[j], sems[j])

    def drain(j, row):
        pltpu.make_async_copy(
            w_hbm.at[idx_v.at[j]], rows_v.at[j], sems[j]).wait()
        pltpu.sync_copy(rows_v.at[j], out_hbm.at[pl.ds(row * 128, 128)])

    loads()
    fire(0)
    fire(1)

    @pl.when(wid < 8)
    def _f2():
        fire(2)

    drain(0, wid)
    drain(1, wid + _NW)

    @pl.when(wid < 8)
    def _d2():
        drain(2, wid + 2 * _NW)


_sc_gather = pl.kernel(
    _gather_body,
    out_type=jax.ShapeDtypeStruct((_N_TOK, _EMBED_DIM), jnp.float32),
    mesh=plsc.VectorSubcoreMesh(
        core_axis_name="c", subcore_axis_name="s",
        num_cores=_NC, num_subcores=_NS),
    scratch_types=[
        pltpu.VMEM((3, 128), jnp.int32),
        pltpu.VMEM((3, 128, _EMBED_DIM), jnp.float32),
        [pltpu.SemaphoreType.DMA] * 3,
    ],
    compiler_params=pltpu.CompilerParams(use_tc_tiling_on_sc=False),
)


@jax.jit
def kernel(z, W):
    z2 = z.reshape(_N_TOK, _EMBED_DIM)
    idx0, idx1 = pl.pallas_call(
        _idx_body,
        grid=(2,),
        in_specs=[
            pl.BlockSpec(memory_space=pl.ANY),
            pl.BlockSpec((_NUM_CODES, _EMBED_DIM), lambda i: (0, 0)),
        ],
        out_specs=[
            pl.BlockSpec((_HROWS, 128), lambda i: (0, 0)),
            pl.BlockSpec((_HROWS, 128), lambda i: (0, 0)),
        ],
        out_shape=[
            jax.ShapeDtypeStruct((_HROWS, 128), jnp.int32),
            jax.ShapeDtypeStruct((_HROWS, 128), jnp.int32),
        ],
        scratch_shapes=[
            pltpu.VMEM((_NUM_CODES, _EMBED_DIM), jnp.float32),
            pltpu.VMEM((_N_TOK, _EMBED_DIM), jnp.float32),
            [pltpu.SemaphoreType.DMA] * 2,
        ],
    )(z2, W)
    return _sc_gather(W, idx0, idx1).reshape(_B, _S, _EMBED_DIM)


# speedup vs baseline: 1.0135x; 1.0135x over previous
import functools

import jax
import jax.numpy as jnp
from jax.experimental import pallas as pl
from jax.experimental.pallas import tpu as pltpu
from jax.experimental.pallas import tpu_sc as plsc

_EMBED_DIM = 64
_NUM_CODES = 1024
_B = 16
_S = 576
_N_TOK = _B * _S
_TILE = _N_TOK // 2
_HROWS = _TILE // 128

_NC = 2
_NS = 16
_NW = _NC * _NS


def _idx_body(z_ref, w_ref, idx0_ref, idx1_ref, wn_ref):
    @pl.when(pl.program_id(0) == 0)
    def _init():
        w = w_ref[...]
        wn_ref[...] = w / jnp.maximum(
            jnp.sqrt(jnp.sum(w * w, axis=1, keepdims=True)), 1e-12)

    zt = z_ref[...]
    zn = zt / jnp.maximum(
        jnp.sqrt(jnp.sum(zt * zt, axis=1, keepdims=True)), 1e-12)
    scores = jax.lax.dot_general(
        zn, wn_ref[...], (((1,), (1,)), ((), ())),
        preferred_element_type=jnp.float32)
    m = jnp.max(scores, axis=1, keepdims=True)
    ids = jax.lax.broadcasted_iota(
        jnp.int32, scores.shape, 1).astype(jnp.float32)
    idx = jnp.min(jnp.where(scores == m, ids, jnp.float32(4096.0)), axis=1)
    packed = idx.astype(jnp.int32).reshape(_HROWS, 128)
    i = pl.program_id(0)

    @pl.when(i == 0)
    def _w0():
        idx0_ref[...] = packed

    @pl.when(i == 1)
    def _w1():
        idx1_ref[...] = packed


def _gather_body(w_hbm, idx0_hbm, idx1_hbm, out_hbm, idx_v, rows_v, sems):
    wid = jax.lax.axis_index("s") * _NC + jax.lax.axis_index("c")

    def loads():
        pltpu.sync_copy(idx0_hbm.at[pl.ds(wid, 1), :], idx_v.at[pl.ds(0, 1)])

        @pl.when(wid < 4)
        def _a():
            pltpu.sync_copy(idx0_hbm.at[pl.ds(wid + 32, 1), :],
                            idx_v.at[pl.ds(1, 1)])

        @pl.when(wid >= 4)
        def _b():
            pltpu.sync_copy(idx1_hbm.at[pl.ds(wid - 4, 1), :],
                            idx_v.at[pl.ds(1, 1)])

        @pl.when(wid < 8)
        def _c():
            pltpu.sync_copy(idx1_hbm.at[pl.ds(wid + 28, 1), :],
                            idx_v.at[pl.ds(2, 1)])

    def fire(j):
        pltpu.async_copy(w_hbm.at[idx_v.at[j]], rows_v.at[j], sems[j])

    def drain(j, row):
        pltpu.make_async_copy(
            w_hbm.at[idx_v.at[j]], rows_v.at[j], sems[j]).wait()
        pltpu.sync_copy(rows_v.at[j], out_hbm.at[pl.ds(row * 128, 128)])

    loads()
    fire(0)
    fire(1)

    @pl.when(wid < 8)
    def _f2():
        fire(2)

    drain(0, wid)
    drain(1, wid + _NW)

    @pl.when(wid < 8)
    def _d2():
        drain(2, wid + 2 * _NW)


_sc_gather = pl.kernel(
    _gather_body,
    out_type=jax.ShapeDtypeStruct((_N_TOK, _EMBED_DIM), jnp.float32),
    mesh=plsc.VectorSubcoreMesh(
        core_axis_name="c", subcore_axis_name="s",
        num_cores=_NC, num_subcores=_NS),
    scratch_types=[
        pltpu.VMEM((3, 128), jnp.int32),
        pltpu.VMEM((3, 128, _EMBED_DIM), jnp.float32),
        [pltpu.SemaphoreType.DMA] * 3,
    ],
    compiler_params=pltpu.CompilerParams(use_tc_tiling_on_sc=False),
)


@jax.jit
def kernel(z, W):
    z2 = z.reshape(_N_TOK, _EMBED_DIM)
    idx0, idx1 = pl.pallas_call(
        _idx_body,
        grid=(2,),
        in_specs=[
            pl.BlockSpec((_TILE, _EMBED_DIM), lambda i: (i, 0)),
            pl.BlockSpec((_NUM_CODES, _EMBED_DIM), lambda i: (0, 0)),
        ],
        out_specs=[
            pl.BlockSpec((_HROWS, 128), lambda i: (0, 0)),
            pl.BlockSpec((_HROWS, 128), lambda i: (0, 0)),
        ],
        out_shape=[
            jax.ShapeDtypeStruct((_HROWS, 128), jnp.int32),
            jax.ShapeDtypeStruct((_HROWS, 128), jnp.int32),
        ],
        scratch_shapes=[pltpu.VMEM((_NUM_CODES, _EMBED_DIM), jnp.float32)],
    )(z2, W)
    return _sc_gather(W, idx0, idx1).reshape(_B, _S, _EMBED_DIM)
